# Initial kernel scaffold; baseline (speedup 1.0000x reference)
import jax
import jax.numpy as jnp
from jax.experimental import pallas as pl


def kernel(x, edge_index, batch, W1, b1, W2, b2, W3, b3, conv1d_w, conv1d_b, fc1_w, fc1_b, fc2_w, fc2_b):
    raise NotImplementedError("write your pallas kernel here")


# speedup vs baseline: 3.2079x; 3.2079x over previous
import dataclasses
import functools

import jax
import jax.numpy as jnp
from jax import lax
from jax.experimental import pallas as pl
from jax.experimental.pallas import tpu as pltpu
from jax.experimental.pallas import tpu_sc as plsc

N = 10000
D = 128
G = 200
K = 30
NC = 2
NS = 16
NW = NC * NS
CHUNK = 128
NPAD = 10240
DUMMY = N
RPT = NPAD // NS
DW = 16
HPAD = N + 16
KS = 32
BM = 1000
NEG = -3.0e38
JUNK = N + 16

_mesh = plsc.VectorSubcoreMesh(core_axis_name="c", subcore_axis_name="s")

_sc_params = pltpu.CompilerParams()
if "needs_layout_passes" in pltpu.CompilerParams.__dataclass_fields__:
    _sc_params = dataclasses.replace(_sc_params, needs_layout_passes=False)


def _zero_vmem_2d(ref, rows, cols):
    @pl.loop(0, rows)
    def _(r):
        @pl.loop(0, cols, step=16)
        def _(c0):
            ref[r, pl.ds(c0, 16)] = jnp.zeros((16,), ref.dtype)


W_WIN = 20640
NWIN = 16
WCH = 128
W_STORE = 20736
NCHW = W_STORE // WCH


def make_sorted_agg_kernel(with_gather):
    def merge_and_done(tgt, midx_v, mbuf, rk, lane_iota, s):
        midx_v[...] = jnp.where(
            lane_iota == 0, rk,
            jnp.where(lane_iota == 1, N + s, jnp.int32(DUMMY)),
        )
        pltpu.sync_copy(tgt.at[midx_v], mbuf)

        @pl.loop(0, D, step=16)
        def _(c0):
            mbuf[0, pl.ds(c0, 16)] = mbuf[0, pl.ds(c0, 16)] + mbuf[1, pl.ds(c0, 16)]

        midx_v[...] = jnp.where(lane_iota == 0, rk, jnp.int32(DUMMY))
        pltpu.sync_copy(mbuf, tgt.at[midx_v])

    if not with_gather:
        @functools.partial(
            pl.kernel,
            out_type=jax.ShapeDtypeStruct((NPAD, D), jnp.float32),
            mesh=_mesh,
            scratch_types=[
                pltpu.VMEM((NCHW, WCH), jnp.int32),
                pltpu.VMEM((1, CHUNK), jnp.int32),
                pltpu.VMEM((CHUNK, D), jnp.float32),
                pltpu.VMEM((16,), jnp.int32),
                pltpu.VMEM((16, D), jnp.float32),
                pltpu.VMEM((1, 128), jnp.int32),
                pltpu.VMEM_SHARED((NPAD, D), jnp.float32),
            ],
        )
        def agg(pk_hbm, rk_hbm, out_hbm,
                pk_v, drow_v, gbuf, midx_v, mbuf, rk_s, acc):
            c = lax.axis_index("c")
            s = lax.axis_index("s")

            @pl.when(c == 0)
            def _():
                lane_iota = lax.iota(jnp.int32, 16)
                pltpu.sync_copy(rk_hbm.at[s], rk_s)
                pltpu.sync_copy(pk_hbm.at[s], pk_v)
                _zero_vmem_2d(gbuf, CHUNK, D)

                @pl.loop(0, RPT // CHUNK)
                def _(b):
                    pltpu.sync_copy(gbuf, acc.at[pl.ds(s * RPT + b * CHUNK, CHUNK)])

                @pl.loop(0, CHUNK)
                def _(r):
                    @pl.loop(0, D, step=16)
                    def _(c0):
                        gbuf[r, pl.ds(c0, 16)] = jnp.ones((16,), jnp.float32)

                plsc.subcore_barrier()

                @pl.loop(0, NCHW)
                def _(j):
                    for c0 in range(0, WCH, 16):
                        v = pk_v[j, pl.ds(c0, 16)]
                        drow_v[0, pl.ds(c0, 16)] = v >> 16
                    pltpu.sync_copy(gbuf, acc.at[drow_v.at[0]], add=True)

                plsc.subcore_barrier()
                merge_and_done(acc, midx_v, mbuf, rk_s[0, pl.ds(0, 16)][0],
                               lane_iota, s)
                plsc.subcore_barrier()
                pltpu.sync_copy(acc.at[pl.ds(s * RPT, RPT)],
                                out_hbm.at[pl.ds(s * RPT, RPT)])

        return agg

    @functools.partial(
        pl.kernel,
        out_type=jax.ShapeDtypeStruct((NPAD, D), jnp.float32),
        mesh=_mesh,
        scratch_types=[
            pltpu.VMEM((W_STORE,), jnp.int32),
            pltpu.VMEM((CHUNK,), jnp.int32),
            pltpu.VMEM((CHUNK, D), jnp.float32),
            pltpu.VMEM((CHUNK, D), jnp.float32),
            pltpu.VMEM((CHUNK,), jnp.int32),
            pltpu.VMEM((16,), jnp.int32),
            pltpu.VMEM((16, D), jnp.float32),
            pltpu.VMEM((NPAD,), jnp.float32),
        ],
        compiler_params=_sc_params,
    )
    def agg(z_hbm, pk_hbm, dinv_hbm, out_hbm,
            pk_v, srow_v, gbuf, fbuf, fidx_v, midx_v, mbuf, dinv_v):
        c = lax.axis_index("c")
        s = lax.axis_index("s")

        @pl.when(c == 0)
        def _():
            lane_iota = lax.iota(jnp.int32, 16)
            pltpu.sync_copy(pk_hbm.at[pl.ds(s * W_STORE, W_STORE)], pk_v)
            pltpu.sync_copy(dinv_hbm, dinv_v)
            for c0 in range(0, CHUNK, 16):
                fidx_v[pl.ds(c0, 16)] = jnp.full((16,), JUNK, jnp.int32)

            zero16 = jnp.zeros((16,), jnp.float32)
            init = (jnp.int32(-1), jnp.int32(0)) + (zero16,) * 8

            def flush_row(acc8, fpos, prev, maskval):
                frow = jnp.full((16,), 0, jnp.int32) + fpos
                for k in range(8):
                    plsc.store_scatter(fbuf, [frow, k * 16 + lane_iota], acc8[k])
                plsc.store_scatter(
                    fidx_v, [frow],
                    jnp.full((16,), 0, jnp.int32) + prev,
                    mask=(lane_iota == maskval))

            def step(dval, sval, prev, fpos, acc8, ridx):
                new = dval != prev
                flushing = jnp.logical_and(new, prev >= 0)
                flush_row(acc8, fpos, prev, jnp.where(flushing, 0, 99))
                fpos = fpos + jnp.where(flushing, 1, 0)

                @pl.when(fpos == CHUNK)
                def _():
                    pltpu.sync_copy(fbuf, out_hbm.at[fidx_v])
                    for c0 in range(0, CHUNK, 16):
                        fidx_v[pl.ds(c0, 16)] = jnp.full((16,), JUNK, jnp.int32)

                fpos = jnp.where(fpos == CHUNK, 0, fpos)
                nv = lane_iota >= jnp.where(new, 0, 16)
                rvec = jnp.full((16,), 0, jnp.int32) + ridx
                norm = (dinv_v[pl.ds(sval, 16)][0]
                        * dinv_v[pl.ds(dval, 16)][0])
                out8 = []
                for k in range(8):
                    rowk = plsc.load_gather(gbuf, [rvec, k * 16 + lane_iota])
                    uk = rowk * norm
                    out8.append(jnp.where(nv, uk, acc8[k] + uk))
                return dval, fpos, out8

            def chunk_body(j, carry):
                for c0 in range(0, WCH, 16):
                    v = pk_v[pl.ds(j * WCH + c0, 16)]
                    srow_v[pl.ds(c0, 16)] = jnp.minimum(v & 0xFFFF, N - 1)
                pltpu.sync_copy(z_hbm.at[srow_v], gbuf)

                def group_body(g, carry2):
                    prev, fpos = carry2[0], carry2[1]
                    acc8 = list(carry2[2:])
                    for l in range(16):
                        v0 = pk_v[pl.ds(j * WCH + g * 16 + l, 16)][0]
                        prev, fpos, acc8 = step(v0 >> 16, v0 & 0xFFFF, prev,
                                                fpos, acc8, g * 16 + l)
                    return (prev, fpos, *acc8)

                return lax.fori_loop(0, WCH // 16, group_body, carry)

            carry = lax.fori_loop(0, NCHW, chunk_body, init)
            prev, fpos = carry[0], carry[1]
            acc8 = list(carry[2:])
            flush_row(acc8, fpos, prev, 0)
            pltpu.sync_copy(fbuf, out_hbm.at[fidx_v])

            plsc.subcore_barrier()
            rkv = pk_v[pl.ds(W_WIN, 16)][0] & 0xFFFF
            midx_v[...] = jnp.where(
                lane_iota == 0, rkv,
                jnp.where(lane_iota == 1, N + s, jnp.int32(DUMMY)),
            )
            pltpu.sync_copy(out_hbm.at[midx_v], mbuf)

            @pl.loop(0, D, step=16)
            def _(c0):
                cvec = c0 + lane_iota
                a = plsc.load_gather(mbuf, [jnp.zeros((16,), jnp.int32), cvec])
                b = plsc.load_gather(mbuf, [jnp.full((16,), 1, jnp.int32), cvec])
                plsc.store_scatter(mbuf, [jnp.zeros((16,), jnp.int32), cvec], a + b)

            midx_v[...] = jnp.where(lane_iota == 0, rkv, jnp.int32(DUMMY))
            pltpu.sync_copy(mbuf, out_hbm.at[midx_v])

    return agg


def make_pool_kernel():
    @functools.partial(
        pl.kernel,
        out_type=jax.ShapeDtypeStruct((G, KS, 3 * D), jnp.float32),
        mesh=_mesh,
        scratch_types=[
            pltpu.VMEM((N,), jnp.float32),
            pltpu.VMEM((N + 16,), jnp.int32),
            pltpu.VMEM((7, KS), jnp.int32),
            pltpu.VMEM((KS, 3 * D), jnp.float32),
        ],
        compiler_params=_sc_params,
    )
    def pool(key_hbm, bat_hbm, hcat_hbm, out_hbm, key_v, bat_v, idx_v, gbuf):
        c = lax.axis_index("c")
        s = lax.axis_index("s")
        w = c * NS + s
        pltpu.sync_copy(key_hbm, key_v)
        pltpu.sync_copy(bat_hbm, bat_v.at[pl.ds(0, N)])

        ng = jnp.where(w < 8, 7, 6)
        g0 = 6 * w + jnp.minimum(w, 8)

        lane_iota = lax.iota(jnp.int32, 16)

        def bat_at(i):
            return bat_v[pl.ds(i, 16)][0]

        def lower_bound(val):
            def body(_, lohi):
                lo, hi = lohi
                mid = (lo + hi) // 2
                go_right = bat_at(mid) < val
                return (jnp.where(go_right, mid + 1, lo), jnp.where(go_right, hi, mid))

            lo, _ = lax.fori_loop(0, 14, body, (jnp.int32(0), jnp.int32(N)))
            return lo

        def graph_body(gi, off):
            g = g0 + gi
            end = lower_bound(g + 1)
            cnt = end - off
            nsel = jnp.minimum(cnt, K)
            base0 = (off // 16) * 16
            nchunks = (end - base0 + 15) // 16

            def select_one():
                def chunk_body(t, carry):
                    m, best = carry
                    c0 = base0 + t * 16
                    lanes = c0 + lane_iota
                    kv = key_v[pl.ds(c0, 16)]
                    mask = (lanes >= off) & (lanes < end)
                    kv2 = jnp.where(mask, kv, NEG)
                    cmax = jnp.max(kv2)
                    cand = jnp.min(jnp.where(kv2 == cmax, lanes, jnp.int32(N)))
                    better = cmax > m
                    return (jnp.maximum(m, cmax), jnp.where(better, cand, best))

                _, best = lax.fori_loop(
                    0, nchunks, chunk_body, (jnp.float32(NEG), jnp.int32(DUMMY))
                )
                cb = (best // 16) * 16
                kv = key_v[pl.ds(cb, 16)]
                key_v[pl.ds(cb, 16)] = jnp.where(
                    cb + lane_iota == best, jnp.float32(NEG), kv
                )
                return best

            def slot_body(sl, carry):
                lo_row, hi_row = carry
                idx = lax.cond(sl < nsel, select_one, lambda: jnp.int32(DUMMY))
                lo_row = jnp.where(lane_iota == sl, idx, lo_row)
                hi_row = jnp.where(lane_iota == sl - 16, idx, hi_row)
                return lo_row, hi_row

            dummy_row = jnp.full((16,), DUMMY, jnp.int32)
            lo_row, hi_row = lax.fori_loop(0, KS, slot_body, (dummy_row, dummy_row))
            idx_v[gi, pl.ds(0, 16)] = lo_row
            idx_v[gi, pl.ds(16, 16)] = hi_row
            pltpu.sync_copy(hcat_hbm.at[idx_v.at[gi]], gbuf)
            pltpu.sync_copy(gbuf, out_hbm.at[g])
            return end

        off0 = lower_bound(g0)

        def outer(gi, off):
            return lax.cond(gi < ng, lambda: graph_body(gi, off), lambda: off)

        lax.fori_loop(0, 7, outer, off0)

    return pool


_P_HIGH = jax.lax.Precision.DEFAULT


def _dinv_body(degp_ref, dinv_ref):
    deg = degp_ref[:, 0:1]
    dinv_ref[...] = jnp.where(
        deg > 0, lax.rsqrt(jnp.maximum(deg, 1e-12)), 0.0)


def _mm1_body(x_ref, w_ref, z_ref):
    z_ref[...] = jnp.dot(x_ref[...], w_ref[...],
                         preferred_element_type=jnp.float32, precision=_P_HIGH)


def _mm_mid_body(p_ref, b_ref, w_ref, h_ref, z_ref):
    h = jnp.maximum(p_ref[...] + b_ref[...], 0.0)
    h_ref[...] = h
    z_ref[...] = jnp.dot(h, w_ref[...], preferred_element_type=jnp.float32,
                         precision=_P_HIGH)


def _mm_last_body(p_ref, b_ref, h_ref, key_ref):
    h = jnp.maximum(p_ref[...] + b_ref[...], 0.0)
    h_ref[...] = h
    key_ref[...] = h[:, D - 1:D]


def _readout_a_body(p2_ref, cwt_ref, cb_ref, m_ref):
    a = p2_ref[:, : 3 * D]
    b = p2_ref[:, 3 * D:]
    ce = jnp.maximum(
        jnp.dot(a, cwt_ref[...], preferred_element_type=jnp.float32, precision=_P_HIGH)
        + cb_ref[...],
        0.0,
    )
    co = jnp.maximum(
        jnp.dot(b, cwt_ref[...], preferred_element_type=jnp.float32, precision=_P_HIGH)
        + cb_ref[...],
        0.0,
    )
    m_ref[...] = jnp.maximum(ce, co)


def _readout_b_body(m2_ref, f1w_ref, f1b_ref, f2w_ref, f2b_ref, o_ref):
    h = jnp.maximum(
        jnp.dot(m2_ref[...], f1w_ref[...], preferred_element_type=jnp.float32,
                precision=_P_HIGH)
        + f1b_ref[...],
        0.0,
    )
    o_ref[...] = (
        jnp.dot(h, f2w_ref[...], preferred_element_type=jnp.float32, precision=_P_HIGH)
        + f2b_ref[...]
    )


def kernel(x, edge_index, batch, W1, b1, W2, b2, W3, b3, conv1d_w, conv1d_b,
           fc1_w, fc1_b, fc2_w, fc2_b):
    n = x.shape[0]
    e_raw = edge_index.shape[1]
    e_tot = e_raw + n
    e16 = NWIN * W_WIN

    loops = jnp.arange(n, dtype=jnp.int32)
    src_all = jnp.concatenate([edge_index[0], loops])
    dst_all = jnp.concatenate([edge_index[1], loops])
    perm = jnp.argsort(dst_all, stable=True)
    ds = jnp.concatenate([dst_all[perm], jnp.full((e16 - e_tot,), DUMMY, jnp.int32)])
    ss = jnp.concatenate([src_all[perm], jnp.zeros((e16 - e_tot,), jnp.int32)])

    bpos = jnp.arange(1, NWIN, dtype=jnp.int32) * W_WIN
    bval = ds[bpos]
    straddle = bval == ds[bpos - 1]
    run_end = jnp.searchsorted(ds, bval, side="right").astype(jnp.int32)
    hk = jnp.where(straddle, run_end - bpos, 0)
    rk = jnp.concatenate(
        [jnp.full((1,), DUMMY, jnp.int32), jnp.where(straddle, bval, DUMMY)]
    )
    rk3 = jnp.broadcast_to(rk[:, None, None], (NWIN, 1, 128)).astype(jnp.int32)
    hk_full = jnp.concatenate([jnp.zeros((1,), jnp.int32), hk])
    pos = jnp.arange(e16, dtype=jnp.int32)
    win = pos // W_WIN
    head = pos < win * W_WIN + hk_full[win]
    ds_mod = jnp.where(head, n + win, ds)

    padw = W_STORE - W_WIN
    pk_flat = (ds_mod << 16) | ss
    pk = jnp.pad(pk_flat.reshape(NWIN, W_WIN), ((0, 0), (0, padw)),
                 constant_values=DUMMY << 16)
    pk = pk.at[:, W_WIN].set((DUMMY << 16) | rk)
    pkflat = pk.reshape(NWIN * W_STORE)
    pk3 = pk.reshape(NWIN, NCHW, WCH)

    agg_k = make_sorted_agg_kernel(True)
    ones_k = make_sorted_agg_kernel(False)
    pool_k = make_pool_kernel()

    degpad = ones_k(pk3, rk3)

    grid = (n // BM,)
    row_spec = pl.BlockSpec((BM, D), lambda i: (i, 0))
    w_spec = pl.BlockSpec((D, D), lambda i: (0, 0))
    b_spec = pl.BlockSpec((1, D), lambda i: (0, 0))
    p_spec = pl.BlockSpec((BM, D), lambda i: (i, 0))
    key_spec = pl.BlockSpec((BM, 1), lambda i: (i, 0))

    BMD = NPAD // 10
    dinv2 = pl.pallas_call(
        _dinv_body,
        grid=(10,),
        in_specs=[pl.BlockSpec((BMD, D), lambda i: (i, 0))],
        out_specs=pl.BlockSpec((BMD, 1), lambda i: (i, 0)),
        out_shape=jax.ShapeDtypeStruct((NPAD, 1), jnp.float32),
    )(degpad)
    dinv1 = dinv2.reshape(NPAD)
    kidx = jnp.arange(1, NWIN, dtype=jnp.int32)
    dinv1 = dinv1.at[n + kidx].set(dinv1[rk[1:]])

    z1 = pl.pallas_call(
        _mm1_body,
        grid=grid,
        in_specs=[row_spec, w_spec],
        out_specs=row_spec,
        out_shape=jax.ShapeDtypeStruct((n, D), jnp.float32),
    )(x, W1)

    p1 = agg_k(z1, pkflat, dinv1)

    h1, z2 = pl.pallas_call(
        _mm_mid_body,
        grid=grid,
        in_specs=[p_spec, b_spec, w_spec],
        out_specs=[row_spec, row_spec],
        out_shape=[
            jax.ShapeDtypeStruct((n, D), jnp.float32),
            jax.ShapeDtypeStruct((n, D), jnp.float32),
        ],
    )(p1, b1.reshape(1, D), W2)

    p2 = agg_k(z2, pkflat, dinv1)

    h2, z3 = pl.pallas_call(
        _mm_mid_body,
        grid=grid,
        in_specs=[p_spec, b_spec, w_spec],
        out_specs=[row_spec, row_spec],
        out_shape=[
            jax.ShapeDtypeStruct((n, D), jnp.float32),
            jax.ShapeDtypeStruct((n, D), jnp.float32),
        ],
    )(p2, b2.reshape(1, D), W3)

    p3 = agg_k(z3, pkflat, dinv1)

    h3, key = pl.pallas_call(
        _mm_last_body,
        grid=grid,
        in_specs=[p_spec, b_spec],
        out_specs=[row_spec, key_spec],
        out_shape=[
            jax.ShapeDtypeStruct((n, D), jnp.float32),
            jax.ShapeDtypeStruct((n, 1), jnp.float32),
        ],
    )(p3, b3.reshape(1, D))

    hcat = jnp.concatenate([h1, h2, h3], axis=1)
    hcat = jnp.concatenate(
        [hcat, jnp.zeros((HPAD - n, 3 * D), jnp.float32)], axis=0
    )

    pooled = pool_k(key.reshape(n), batch, hcat)

    pooled2 = pooled[:, :K, :].reshape(G * K // 2, 6 * D)
    cwt = conv1d_w[:, 0, :].T

    m = pl.pallas_call(
        _readout_a_body,
        in_specs=[
            pl.BlockSpec((G * K // 2, 6 * D), lambda: (0, 0)),
            pl.BlockSpec((3 * D, 16), lambda: (0, 0)),
            pl.BlockSpec((1, 16), lambda: (0, 0)),
        ],
        out_specs=pl.BlockSpec((G * K // 2, 16), lambda: (0, 0)),
        out_shape=jax.ShapeDtypeStruct((G * K // 2, 16), jnp.float32),
    )(pooled2, cwt, conv1d_b.reshape(1, 16))

    m2 = m.reshape(G, K // 2 * 16)
    f1p = fc1_w.reshape(16, K // 2, fc1_w.shape[1]).transpose(1, 0, 2).reshape(
        K // 2 * 16, fc1_w.shape[1]
    )

    out = pl.pallas_call(
        _readout_b_body,
        in_specs=[
            pl.BlockSpec((G, K // 2 * 16), lambda: (0, 0)),
            pl.BlockSpec((K // 2 * 16, 128), lambda: (0, 0)),
            pl.BlockSpec((1, 128), lambda: (0, 0)),
            pl.BlockSpec((128, 1), lambda: (0, 0)),
            pl.BlockSpec((1, 1), lambda: (0, 0)),
        ],
        out_specs=pl.BlockSpec((G, 1), lambda: (0, 0)),
        out_shape=jax.ShapeDtypeStruct((G, 1), jnp.float32),
    )(m2, f1p, fc1_b.reshape(1, 128), fc2_w, fc2_b.reshape(1, 1))

    return out
